# Initial kernel scaffold; baseline (speedup 1.0000x reference)
import jax
import jax.numpy as jnp
from jax.experimental import pallas as pl


def kernel(input, adj_indices, adj_values, adj_ned_indices, adj_ned_values, Wg1, Wg2, W1, b1, W2, b2):
    raise NotImplementedError("write your pallas kernel here")


# speedup vs baseline: 15.5805x; 15.5805x over previous
import dataclasses
import functools

import jax
import jax.numpy as jnp
from jax import lax
from jax.experimental import pallas as pl
from jax.experimental.pallas import tpu as pltpu
from jax.experimental.pallas import tpu_sc as plsc

_N = 2048
_F = 256
_K = 32
_E = 32768
_NC = 2
_NS = 16
_NW = _NC * _NS
_SLAB = _N // _NW // 2
_ECH = 8192
_BLK = 256
_G = _N // _BLK

_HI = jax.lax.Precision.HIGHEST
_P_SIM = jax.lax.Precision.DEFAULT
_P_ADJ = jax.lax.Precision.DEFAULT


def _dot(a, b, trans_b=False, prec=_HI):
    dn = (((1,), (1 if trans_b else 0,)), ((), ()))
    return lax.dot_general(a, b, dn, precision=prec,
                           preferred_element_type=jnp.float32)


def _sc_scatter_body(r1, c1, v1, r2, c2, v2, o1, o2, d1, d2,
                     slab, dslab, er, ec, ev):
    wid = lax.axis_index("s") * _NC + lax.axis_index("c")
    zeros16 = jnp.zeros((16,), jnp.float32)
    for (rh, ch, vh, oh, dh) in ((r1, c1, v1, o1, d1), (r2, c2, v2, o2, d2)):
        for sl in range(2):
            base = wid * (2 * _SLAB) + sl * _SLAB

            @pl.loop(0, _SLAB * _N // 16)
            def _(i):
                slab[pl.ds(i * 16, 16)] = zeros16

            @pl.loop(0, _SLAB // 16)
            def _(i):
                dslab[pl.ds(i * 16, 16)] = zeros16

            for chk in range(_E // _ECH):
                pltpu.sync_copy(rh.at[pl.ds(chk * _ECH, _ECH)], er)
                pltpu.sync_copy(ch.at[pl.ds(chk * _ECH, _ECH)], ec)
                pltpu.sync_copy(vh.at[pl.ds(chk * _ECH, _ECH)], ev)

                @pl.loop(0, _ECH // 16)
                def _(i):
                    r = er[pl.ds(i * 16, 16)]
                    c = ec[pl.ds(i * 16, 16)]
                    v = ev[pl.ds(i * 16, 16)]
                    rel = r - base
                    mask = (rel >= 0) & (rel < _SLAB)
                    idx = jnp.where(mask, rel * _N + c, 0)
                    vv = jnp.where(mask, v, 0.0)
                    plsc.addupdate_scatter(slab, [idx], vv, mask=mask)
                    didx = jnp.where(mask, rel, 0)
                    plsc.addupdate_scatter(dslab, [didx], vv, mask=mask)

            pltpu.sync_copy(slab, oh.at[pl.ds(base * _N, _SLAB * _N)])
            pltpu.sync_copy(dslab, dh.at[pl.ds(base, _SLAB)])


def _densify_pair(adj_indices, adj_values, ned_indices, ned_values):
    cp = pltpu.CompilerParams()
    if "needs_layout_passes" in pltpu.CompilerParams.__dataclass_fields__:
        cp = dataclasses.replace(cp, needs_layout_passes=False)
    f = pl.kernel(
        _sc_scatter_body,
        out_type=(jax.ShapeDtypeStruct((_N * _N,), jnp.float32),
                  jax.ShapeDtypeStruct((_N * _N,), jnp.float32),
                  jax.ShapeDtypeStruct((_N,), jnp.float32),
                  jax.ShapeDtypeStruct((_N,), jnp.float32)),
        mesh=plsc.VectorSubcoreMesh(core_axis_name="c", subcore_axis_name="s"),
        scratch_types=[pltpu.VMEM((_SLAB * _N,), jnp.float32),
                       pltpu.VMEM((_SLAB,), jnp.float32),
                       pltpu.VMEM((_ECH,), jnp.int32),
                       pltpu.VMEM((_ECH,), jnp.int32),
                       pltpu.VMEM((_ECH,), jnp.float32)],
        compiler_params=cp,
    )
    o1, o2, d1, d2 = f(adj_indices[0], adj_indices[1], adj_values,
                       ned_indices[0], ned_indices[1], ned_values)
    return o1.reshape(_N, _N), o2.reshape(_N, _N), d1, d2


def _embed1_body(a_ref, x_ref, wg1_ref, dega_ref, degab_ref, ne1_ref):
    s = 1.0 / (jnp.sqrt(dega_ref[...]) + 1e-10)
    s_blk = 1.0 / (jnp.sqrt(degab_ref[...]) + 1e-10)
    ys = x_ref[...] * wg1_ref[...][None, :] * s[:, None]
    ne1_ref[...] = jnp.tanh(_dot(a_ref[...], ys) * s_blk[:, None])


def _embed2_body(a_ref, ne1_ref, wg2_ref, dega_ref, degab_ref, ne_ref):
    s = 1.0 / (jnp.sqrt(dega_ref[...]) + 1e-10)
    s_blk = 1.0 / (jnp.sqrt(degab_ref[...]) + 1e-10)
    y2 = ne1_ref[...] * wg2_ref[...][None, :] * s[:, None]
    ne2 = _dot(a_ref[...], y2) * s_blk[:, None]
    nrm = jnp.sqrt(jnp.sum(ne2 * ne2, axis=1, keepdims=True))
    ne_ref[...] = ne2 / jnp.maximum(nrm, 1e-12)


def _sim_body(neb_ref, ne_ref, sim_ref):
    sim_ref[...] = _dot(neb_ref[...], ne_ref[...], trans_b=True, prec=_P_SIM)


def _topk_body(an_ref, degn_ref, sim_ref, m1_ref):
    inv_blk = 1.0 / (degn_ref[...] + 1e-10)
    W0 = _dot(inv_blk[:, None] * an_ref[...], sim_ref[...], prec=_P_ADJ)
    W = W0
    for _ in range(_K - 1):
        m = jnp.max(W, axis=1, keepdims=True)
        W = jnp.where(W == m, -jnp.inf, W)
    t = jnp.max(W, axis=1)
    m1_ref[...] = jnp.where(W0 >= t[:, None], W0, 0.0)


def _fin_body(araw_ref, m1_ref, m1c_ref, afin_ref, degf_ref):
    af = araw_ref[...] + m1_ref[...] + jnp.transpose(m1c_ref[...], (1, 0))
    afin_ref[...] = af
    degf_ref[...] = jnp.sum(af, axis=1)


def _mid_body(degf_ref, x_ref, w1_ref, xs_ref):
    sf = 1.0 / (jnp.sqrt(degf_ref[...]) + 1e-10)
    xs_ref[...] = sf[:, None] * _dot(x_ref[...], w1_ref[...])


def _gcn1_body(afin_ref, degf_ref, xs_ref, b1_ref, h_ref):
    sf_blk = 1.0 / (jnp.sqrt(degf_ref[...]) + 1e-10)
    h_ref[...] = jnp.maximum(
        sf_blk[:, None] * _dot(afin_ref[...], xs_ref[...]) + b1_ref[...][None, :], 0.0)


def _gcn2_body(afin_ref, degf_ref, degfb_ref, h_ref, w2_ref, b2_ref, out_ref):
    sf = 1.0 / (jnp.sqrt(degf_ref[...]) + 1e-10)
    sf_blk = 1.0 / (jnp.sqrt(degfb_ref[...]) + 1e-10)
    hs = sf[:, None] * _dot(h_ref[...], w2_ref[...])
    out_ref[...] = sf_blk[:, None] * _dot(afin_ref[...], hs) + b2_ref[...][None, :]


def _row_blk(i):
    return (i, 0)


def _full2(i):
    return (0, 0)


def _full1(i):
    return (0,)


def kernel(input, adj_indices, adj_values, adj_ned_indices, adj_ned_values,
           Wg1, Wg2, W1, b1, W2, b2):
    A_raw, A_ned, deg_a, deg_n = _densify_pair(
        adj_indices, adj_values, adj_ned_indices, adj_ned_values)

    f32 = jnp.float32
    row_spec = pl.BlockSpec((_BLK, _N), _row_blk)
    vec_blk = pl.BlockSpec((_BLK,), lambda i: (i,))
    ne_full = pl.BlockSpec((_N, _F), _full2)
    vec_full = pl.BlockSpec((_N,), _full1)

    ne1 = pl.pallas_call(
        _embed1_body, grid=(_G,),
        in_specs=[row_spec, ne_full, pl.BlockSpec((_F,), _full1), vec_full,
                  vec_blk],
        out_specs=pl.BlockSpec((_BLK, _F), _row_blk),
        out_shape=jax.ShapeDtypeStruct((_N, _F), f32),
    )(A_raw, input, Wg1, deg_a, deg_a)

    ne = pl.pallas_call(
        _embed2_body, grid=(_G,),
        in_specs=[row_spec, ne_full, pl.BlockSpec((_F,), _full1), vec_full,
                  vec_blk],
        out_specs=pl.BlockSpec((_BLK, _F), _row_blk),
        out_shape=jax.ShapeDtypeStruct((_N, _F), f32),
    )(A_raw, ne1, Wg2, deg_a, deg_a)

    sim = pl.pallas_call(
        _sim_body, grid=(_G,),
        in_specs=[pl.BlockSpec((_BLK, _F), _row_blk), ne_full],
        out_specs=row_spec,
        out_shape=jax.ShapeDtypeStruct((_N, _N), f32),
    )(ne, ne)

    M1 = pl.pallas_call(
        _topk_body, grid=(_G,),
        in_specs=[row_spec, vec_blk, pl.BlockSpec((_N, _N), _full2)],
        out_specs=row_spec,
        out_shape=jax.ShapeDtypeStruct((_N, _N), f32),
    )(A_ned, deg_n, sim)

    A_fin, deg_f = pl.pallas_call(
        _fin_body, grid=(_G,),
        in_specs=[row_spec, row_spec,
                  pl.BlockSpec((_N, _BLK), lambda i: (0, i))],
        out_specs=(row_spec, pl.BlockSpec((_BLK,), lambda i: (i,))),
        out_shape=(jax.ShapeDtypeStruct((_N, _N), f32),
                   jax.ShapeDtypeStruct((_N,), f32)),
    )(A_raw, M1, M1)

    xs = pl.pallas_call(
        _mid_body,
        out_shape=jax.ShapeDtypeStruct((_N, W1.shape[1]), f32),
    )(deg_f, input, W1)

    h = pl.pallas_call(
        _gcn1_body, grid=(_G,),
        in_specs=[row_spec, vec_blk, pl.BlockSpec((_N, W1.shape[1]), _full2),
                  pl.BlockSpec((W1.shape[1],), _full1)],
        out_specs=pl.BlockSpec((_BLK, W1.shape[1]), _row_blk),
        out_shape=jax.ShapeDtypeStruct((_N, W1.shape[1]), f32),
    )(A_fin, deg_f, xs, b1)

    out = pl.pallas_call(
        _gcn2_body, grid=(_G,),
        in_specs=[row_spec, vec_full, vec_blk,
                  pl.BlockSpec((_N, W1.shape[1]), _full2),
                  pl.BlockSpec((W1.shape[1], b2.shape[0]), _full2),
                  pl.BlockSpec((b2.shape[0],), _full1)],
        out_specs=pl.BlockSpec((_BLK, b2.shape[0]), _row_blk),
        out_shape=jax.ShapeDtypeStruct((_N, b2.shape[0]), f32),
    )(A_fin, deg_f, deg_f, h, W2, b2)
    return out


# speedup vs baseline: 20.2692x; 1.3009x over previous
import dataclasses
import functools

import jax
import jax.numpy as jnp
from jax import lax
from jax.experimental import pallas as pl
from jax.experimental.pallas import tpu as pltpu
from jax.experimental.pallas import tpu_sc as plsc

_N = 2048
_F = 256
_K = 32
_E = 32768
_NC = 2
_NS = 16
_NW = _NC * _NS
_SLAB = _N // _NW // 2
_ECH = 8192
_BLK = 256
_G = _N // _BLK

_HI = jax.lax.Precision.HIGHEST
_P_SIM = jax.lax.Precision.DEFAULT
_P_ADJ = jax.lax.Precision.DEFAULT


def _dot(a, b, trans_b=False, prec=_HI):
    dn = (((1,), (1 if trans_b else 0,)), ((), ()))
    return lax.dot_general(a, b, dn, precision=prec,
                           preferred_element_type=jnp.float32)


def _sc_scatter_body(r1, c1, v1, r2, c2, v2, o1, o2, d1, d2,
                     slab, dslab, er0, ec0, ev0, er1, ec1, ev1, sem0, sem1):
    wid = lax.axis_index("s") * _NC + lax.axis_index("c")
    zeros16 = jnp.zeros((16,), jnp.float32)
    bufs = ((er0, ec0, ev0, sem0), (er1, ec1, ev1, sem1))
    nchunk = _E // _ECH
    for (rh, ch, vh, oh, dh) in ((r1, c1, v1, o1, d1), (r2, c2, v2, o2, d2)):
        for sl in range(2):
            base = wid * (2 * _SLAB) + sl * _SLAB

            def start(chk):
                eb, cb, vb, sm = bufs[chk % 2]
                return (pltpu.async_copy(rh.at[pl.ds(chk * _ECH, _ECH)], eb, sm),
                        pltpu.async_copy(ch.at[pl.ds(chk * _ECH, _ECH)], cb, sm),
                        pltpu.async_copy(vh.at[pl.ds(chk * _ECH, _ECH)], vb, sm))

            hs = start(0)

            @pl.loop(0, _SLAB * _N // 16, unroll=8)
            def _(i):
                slab[pl.ds(i * 16, 16)] = zeros16

            @pl.loop(0, _SLAB // 16)
            def _(i):
                dslab[pl.ds(i * 16, 16)] = zeros16

            for chk in range(nchunk):
                nxt = start(chk + 1) if chk + 1 < nchunk else None
                for h in hs:
                    h.wait()
                eb, cb, vb, _sm = bufs[chk % 2]

                @pl.loop(0, _ECH // 16, unroll=8)
                def _(i):
                    r = eb[pl.ds(i * 16, 16)]
                    c = cb[pl.ds(i * 16, 16)]
                    v = vb[pl.ds(i * 16, 16)]
                    rel = r - base
                    mask = (rel >= 0) & (rel < _SLAB)
                    plsc.addupdate_scatter(slab, [rel * _N + c], v, mask=mask)
                    plsc.addupdate_scatter(dslab, [rel], v, mask=mask)

                hs = nxt

            pltpu.sync_copy(slab, oh.at[pl.ds(base * _N, _SLAB * _N)])
            pltpu.sync_copy(dslab, dh.at[pl.ds(base, _SLAB)])


def _densify_pair(adj_indices, adj_values, ned_indices, ned_values):
    cp = pltpu.CompilerParams()
    if "needs_layout_passes" in pltpu.CompilerParams.__dataclass_fields__:
        cp = dataclasses.replace(cp, needs_layout_passes=False)
    f = pl.kernel(
        _sc_scatter_body,
        out_type=(jax.ShapeDtypeStruct((_N * _N,), jnp.float32),
                  jax.ShapeDtypeStruct((_N * _N,), jnp.float32),
                  jax.ShapeDtypeStruct((_N,), jnp.float32),
                  jax.ShapeDtypeStruct((_N,), jnp.float32)),
        mesh=plsc.VectorSubcoreMesh(core_axis_name="c", subcore_axis_name="s"),
        scratch_types=[pltpu.VMEM((_SLAB * _N,), jnp.float32),
                       pltpu.VMEM((_SLAB,), jnp.float32),
                       pltpu.VMEM((_ECH,), jnp.int32),
                       pltpu.VMEM((_ECH,), jnp.int32),
                       pltpu.VMEM((_ECH,), jnp.float32),
                       pltpu.VMEM((_ECH,), jnp.int32),
                       pltpu.VMEM((_ECH,), jnp.int32),
                       pltpu.VMEM((_ECH,), jnp.float32),
                       pltpu.SemaphoreType.DMA,
                       pltpu.SemaphoreType.DMA],
        compiler_params=cp,
    )
    o1, o2, d1, d2 = f(adj_indices[0], adj_indices[1], adj_values,
                       ned_indices[0], ned_indices[1], ned_values)
    return o1.reshape(_N, _N), o2.reshape(_N, _N), d1, d2


def _embed1_body(a_ref, x_ref, wg1_ref, dega_ref, degab_ref, ne1_ref):
    s = 1.0 / (jnp.sqrt(dega_ref[...]) + 1e-10)
    s_blk = 1.0 / (jnp.sqrt(degab_ref[...]) + 1e-10)
    ys = x_ref[...] * wg1_ref[...][None, :] * s[:, None]
    ne1_ref[...] = jnp.tanh(_dot(a_ref[...], ys) * s_blk[:, None])


def _embed2_body(a_ref, ne1_ref, wg2_ref, dega_ref, degab_ref, ne_ref):
    s = 1.0 / (jnp.sqrt(dega_ref[...]) + 1e-10)
    s_blk = 1.0 / (jnp.sqrt(degab_ref[...]) + 1e-10)
    y2 = ne1_ref[...] * wg2_ref[...][None, :] * s[:, None]
    ne2 = _dot(a_ref[...], y2) * s_blk[:, None]
    nrm = jnp.sqrt(jnp.sum(ne2 * ne2, axis=1, keepdims=True))
    ne_ref[...] = ne2 / jnp.maximum(nrm, 1e-12)


def _sim_body(neb_ref, ne_ref, sim_ref):
    sim_ref[...] = _dot(neb_ref[...], ne_ref[...], trans_b=True, prec=_P_SIM)


def _topk_body(an_ref, degn_ref, sim_ref, m1_ref):
    inv_blk = 1.0 / (degn_ref[...] + 1e-10)
    W0 = _dot(inv_blk[:, None] * an_ref[...], sim_ref[...], prec=_P_ADJ)
    W = W0
    for _ in range(_K - 1):
        m = jnp.max(W, axis=1, keepdims=True)
        W = jnp.where(W == m, -jnp.inf, W)
    t = jnp.max(W, axis=1)
    m1_ref[...] = jnp.where(W0 >= t[:, None], W0, 0.0)


def _fin_body(araw_ref, m1_ref, m1c_ref, afin_ref, degf_ref):
    af = araw_ref[...] + m1_ref[...] + jnp.transpose(m1c_ref[...], (1, 0))
    afin_ref[...] = af
    degf_ref[...] = jnp.sum(af, axis=1)


def _mid_body(degf_ref, x_ref, w1_ref, xs_ref):
    sf = 1.0 / (jnp.sqrt(degf_ref[...]) + 1e-10)
    xs_ref[...] = sf[:, None] * _dot(x_ref[...], w1_ref[...])


def _gcn1_body(afin_ref, degf_ref, xs_ref, b1_ref, h_ref):
    sf_blk = 1.0 / (jnp.sqrt(degf_ref[...]) + 1e-10)
    h_ref[...] = jnp.maximum(
        sf_blk[:, None] * _dot(afin_ref[...], xs_ref[...]) + b1_ref[...][None, :], 0.0)


def _gcn2_body(afin_ref, degf_ref, degfb_ref, h_ref, w2_ref, b2_ref, out_ref):
    sf = 1.0 / (jnp.sqrt(degf_ref[...]) + 1e-10)
    sf_blk = 1.0 / (jnp.sqrt(degfb_ref[...]) + 1e-10)
    hs = sf[:, None] * _dot(h_ref[...], w2_ref[...])
    out_ref[...] = sf_blk[:, None] * _dot(afin_ref[...], hs) + b2_ref[...][None, :]


def _row_blk(i):
    return (i, 0)


def _full2(i):
    return (0, 0)


def _full1(i):
    return (0,)


def kernel(input, adj_indices, adj_values, adj_ned_indices, adj_ned_values,
           Wg1, Wg2, W1, b1, W2, b2):
    A_raw, A_ned, deg_a, deg_n = _densify_pair(
        adj_indices, adj_values, adj_ned_indices, adj_ned_values)

    f32 = jnp.float32
    row_spec = pl.BlockSpec((_BLK, _N), _row_blk)
    vec_blk = pl.BlockSpec((_BLK,), lambda i: (i,))
    ne_full = pl.BlockSpec((_N, _F), _full2)
    vec_full = pl.BlockSpec((_N,), _full1)

    ne1 = pl.pallas_call(
        _embed1_body, grid=(_G,),
        in_specs=[row_spec, ne_full, pl.BlockSpec((_F,), _full1), vec_full,
                  vec_blk],
        out_specs=pl.BlockSpec((_BLK, _F), _row_blk),
        out_shape=jax.ShapeDtypeStruct((_N, _F), f32),
    )(A_raw, input, Wg1, deg_a, deg_a)

    ne = pl.pallas_call(
        _embed2_body, grid=(_G,),
        in_specs=[row_spec, ne_full, pl.BlockSpec((_F,), _full1), vec_full,
                  vec_blk],
        out_specs=pl.BlockSpec((_BLK, _F), _row_blk),
        out_shape=jax.ShapeDtypeStruct((_N, _F), f32),
    )(A_raw, ne1, Wg2, deg_a, deg_a)

    sim = pl.pallas_call(
        _sim_body, grid=(_G,),
        in_specs=[pl.BlockSpec((_BLK, _F), _row_blk), ne_full],
        out_specs=row_spec,
        out_shape=jax.ShapeDtypeStruct((_N, _N), f32),
    )(ne, ne)

    M1 = pl.pallas_call(
        _topk_body, grid=(_G,),
        in_specs=[row_spec, vec_blk, pl.BlockSpec((_N, _N), _full2)],
        out_specs=row_spec,
        out_shape=jax.ShapeDtypeStruct((_N, _N), f32),
    )(A_ned, deg_n, sim)

    A_fin, deg_f = pl.pallas_call(
        _fin_body, grid=(_G,),
        in_specs=[row_spec, row_spec,
                  pl.BlockSpec((_N, _BLK), lambda i: (0, i))],
        out_specs=(row_spec, pl.BlockSpec((_BLK,), lambda i: (i,))),
        out_shape=(jax.ShapeDtypeStruct((_N, _N), f32),
                   jax.ShapeDtypeStruct((_N,), f32)),
    )(A_raw, M1, M1)

    xs = pl.pallas_call(
        _mid_body,
        out_shape=jax.ShapeDtypeStruct((_N, W1.shape[1]), f32),
    )(deg_f, input, W1)

    h = pl.pallas_call(
        _gcn1_body, grid=(_G,),
        in_specs=[row_spec, vec_blk, pl.BlockSpec((_N, W1.shape[1]), _full2),
                  pl.BlockSpec((W1.shape[1],), _full1)],
        out_specs=pl.BlockSpec((_BLK, W1.shape[1]), _row_blk),
        out_shape=jax.ShapeDtypeStruct((_N, W1.shape[1]), f32),
    )(A_fin, deg_f, xs, b1)

    out = pl.pallas_call(
        _gcn2_body, grid=(_G,),
        in_specs=[row_spec, vec_full, vec_blk,
                  pl.BlockSpec((_N, W1.shape[1]), _full2),
                  pl.BlockSpec((W1.shape[1], b2.shape[0]), _full2),
                  pl.BlockSpec((b2.shape[0],), _full1)],
        out_specs=pl.BlockSpec((_BLK, b2.shape[0]), _row_blk),
        out_shape=jax.ShapeDtypeStruct((_N, b2.shape[0]), f32),
    )(A_fin, deg_f, deg_f, h, W2, b2)
    return out


# speedup vs baseline: 27.5825x; 1.3608x over previous
import dataclasses
import functools

import jax
import jax.numpy as jnp
from jax import lax
from jax.experimental import pallas as pl
from jax.experimental.pallas import tpu as pltpu
from jax.experimental.pallas import tpu_sc as plsc

_N = 2048
_F = 256
_K = 32
_E = 32768
_NC = 2
_NS = 16
_NW = _NC * _NS
_SLAB = _N // _NW // 2
_ECH = 8192
_BLK = 256
_G = _N // _BLK

_HI = jax.lax.Precision.HIGHEST
_P_SIM = jax.lax.Precision.DEFAULT
_P_ADJ = jax.lax.Precision.DEFAULT
_P_GCN = jax.lax.Precision.DEFAULT


def _dot(a, b, trans_b=False, prec=_HI):
    dn = (((1,), (1 if trans_b else 0,)), ((), ()))
    return lax.dot_general(a, b, dn, precision=prec,
                           preferred_element_type=jnp.float32)


def _sc_scatter_body(rh, ch, vh, oh, dh,
                     slab, dslab, er0, ec0, ev0, er1, ec1, ev1, sem0, sem1):
    wid = lax.axis_index("s") * _NC + lax.axis_index("c")
    zeros16 = jnp.zeros((16,), jnp.float32)
    bufs = ((er0, ec0, ev0, sem0), (er1, ec1, ev1, sem1))
    nchunk = _E // _ECH
    for sl in range(2):
        base = wid * (2 * _SLAB) + sl * _SLAB

        def start(chk):
            eb, cb, vb, sm = bufs[chk % 2]
            return (pltpu.async_copy(rh.at[pl.ds(chk * _ECH, _ECH)], eb, sm),
                    pltpu.async_copy(ch.at[pl.ds(chk * _ECH, _ECH)], cb, sm),
                    pltpu.async_copy(vh.at[pl.ds(chk * _ECH, _ECH)], vb, sm))

        hs = start(0)

        @pl.loop(0, _SLAB)
        def _(r):
            @pl.loop(0, _N // 16, unroll=8)
            def _(i):
                slab[r, pl.ds(i * 16, 16)] = zeros16

        @pl.loop(0, _SLAB // 16)
        def _(i):
            dslab[pl.ds(i * 16, 16)] = zeros16

        for chk in range(nchunk):
            nxt = start(chk + 1) if chk + 1 < nchunk else None
            for h in hs:
                h.wait()
            eb, cb, vb, _sm = bufs[chk % 2]

            @pl.loop(0, _ECH // 16, unroll=8)
            def _(i):
                r = eb[pl.ds(i * 16, 16)]
                c = cb[pl.ds(i * 16, 16)]
                v = vb[pl.ds(i * 16, 16)]
                rel = r - base
                mask = (rel >= 0) & (rel < _SLAB)
                plsc.addupdate_scatter(slab, [rel, c], v, mask=mask)
                plsc.addupdate_scatter(dslab, [rel], v, mask=mask)

            hs = nxt

        pltpu.sync_copy(slab, oh.at[pl.ds(base, _SLAB)])
        pltpu.sync_copy(dslab, dh.at[pl.ds(base, _SLAB)])


def _densify_one(rows, cols, values):
    cp = pltpu.CompilerParams()
    if "needs_layout_passes" in pltpu.CompilerParams.__dataclass_fields__:
        cp = dataclasses.replace(cp, needs_layout_passes=False)
    f = pl.kernel(
        _sc_scatter_body,
        out_type=(jax.ShapeDtypeStruct((_N, _N), jnp.float32),
                  jax.ShapeDtypeStruct((_N,), jnp.float32)),
        mesh=plsc.VectorSubcoreMesh(core_axis_name="c", subcore_axis_name="s"),
        scratch_types=[pltpu.VMEM((_SLAB, _N), jnp.float32),
                       pltpu.VMEM((_SLAB,), jnp.float32),
                       pltpu.VMEM((_ECH,), jnp.int32),
                       pltpu.VMEM((_ECH,), jnp.int32),
                       pltpu.VMEM((_ECH,), jnp.float32),
                       pltpu.VMEM((_ECH,), jnp.int32),
                       pltpu.VMEM((_ECH,), jnp.int32),
                       pltpu.VMEM((_ECH,), jnp.float32),
                       pltpu.SemaphoreType.DMA,
                       pltpu.SemaphoreType.DMA],
        compiler_params=cp,
    )
    return f(rows, cols, values)


def _densify_pair(adj_indices, adj_values, ned_indices, ned_values):
    o1, d1 = _densify_one(adj_indices[0], adj_indices[1], adj_values)
    o2, d2 = _densify_one(ned_indices[0], ned_indices[1], ned_values)
    return o1, o2, d1, d2


def _embed1_body(a_ref, x_ref, wg1_ref, dega_ref, degab_ref, ne1_ref):
    s = 1.0 / (jnp.sqrt(dega_ref[...]) + 1e-10)
    s_blk = 1.0 / (jnp.sqrt(degab_ref[...]) + 1e-10)
    ys = x_ref[...] * wg1_ref[...][None, :] * s[:, None]
    ne1_ref[...] = jnp.tanh(_dot(a_ref[...], ys) * s_blk[:, None])


def _embed2_body(a_ref, ne1_ref, wg2_ref, dega_ref, degab_ref, ne_ref):
    s = 1.0 / (jnp.sqrt(dega_ref[...]) + 1e-10)
    s_blk = 1.0 / (jnp.sqrt(degab_ref[...]) + 1e-10)
    y2 = ne1_ref[...] * wg2_ref[...][None, :] * s[:, None]
    ne2 = _dot(a_ref[...], y2) * s_blk[:, None]
    nrm = jnp.sqrt(jnp.sum(ne2 * ne2, axis=1, keepdims=True))
    ne_ref[...] = ne2 / jnp.maximum(nrm, 1e-12)


def _sim_body(neb_ref, ne_ref, sim_ref):
    sim_ref[...] = _dot(neb_ref[...], ne_ref[...], trans_b=True, prec=_P_SIM)


def _topk_body(an_ref, degn_ref, sim_ref, m1_ref):
    inv_blk = 1.0 / (degn_ref[...] + 1e-10)
    W0 = _dot(inv_blk[:, None] * an_ref[...], sim_ref[...], prec=_P_ADJ)
    W = W0
    for _ in range(_K - 1):
        m = jnp.max(W, axis=1, keepdims=True)
        W = jnp.where(W == m, -jnp.inf, W)
    t = jnp.max(W, axis=1)
    m1_ref[...] = jnp.where(W0 >= t[:, None], W0, 0.0)


def _fin_body(araw_ref, m1_ref, m1c_ref, afin_ref, degf_ref):
    af = araw_ref[...] + m1_ref[...] + jnp.transpose(m1c_ref[...], (1, 0))
    afin_ref[...] = af
    degf_ref[...] = jnp.sum(af, axis=1)


def _mid_body(degf_ref, x_ref, w1_ref, xs_ref):
    sf = 1.0 / (jnp.sqrt(degf_ref[...]) + 1e-10)
    xs_ref[...] = sf[:, None] * _dot(x_ref[...], w1_ref[...], prec=_P_GCN)


def _gcn1_body(afin_ref, degf_ref, xs_ref, b1_ref, h_ref):
    sf_blk = 1.0 / (jnp.sqrt(degf_ref[...]) + 1e-10)
    h_ref[...] = jnp.maximum(
        sf_blk[:, None] * _dot(afin_ref[...], xs_ref[...], prec=_P_GCN) + b1_ref[...][None, :], 0.0)


def _gcn2_body(afin_ref, degf_ref, degfb_ref, h_ref, w2_ref, b2_ref, out_ref):
    sf = 1.0 / (jnp.sqrt(degf_ref[...]) + 1e-10)
    sf_blk = 1.0 / (jnp.sqrt(degfb_ref[...]) + 1e-10)
    hs = sf[:, None] * _dot(h_ref[...], w2_ref[...], prec=_P_GCN)
    out_ref[...] = sf_blk[:, None] * _dot(afin_ref[...], hs, prec=_P_GCN) + b2_ref[...][None, :]


def _row_blk(i):
    return (i, 0)


def _full2(i):
    return (0, 0)


def _full1(i):
    return (0,)


def kernel(input, adj_indices, adj_values, adj_ned_indices, adj_ned_values,
           Wg1, Wg2, W1, b1, W2, b2):
    A_raw, A_ned, deg_a, deg_n = _densify_pair(
        adj_indices, adj_values, adj_ned_indices, adj_ned_values)

    f32 = jnp.float32
    row_spec = pl.BlockSpec((_BLK, _N), _row_blk)
    vec_blk = pl.BlockSpec((_BLK,), lambda i: (i,))
    ne_full = pl.BlockSpec((_N, _F), _full2)
    vec_full = pl.BlockSpec((_N,), _full1)

    ne1 = pl.pallas_call(
        _embed1_body, grid=(_G,),
        in_specs=[row_spec, ne_full, pl.BlockSpec((_F,), _full1), vec_full,
                  vec_blk],
        out_specs=pl.BlockSpec((_BLK, _F), _row_blk),
        out_shape=jax.ShapeDtypeStruct((_N, _F), f32),
    )(A_raw, input, Wg1, deg_a, deg_a)

    ne = pl.pallas_call(
        _embed2_body, grid=(_G,),
        in_specs=[row_spec, ne_full, pl.BlockSpec((_F,), _full1), vec_full,
                  vec_blk],
        out_specs=pl.BlockSpec((_BLK, _F), _row_blk),
        out_shape=jax.ShapeDtypeStruct((_N, _F), f32),
    )(A_raw, ne1, Wg2, deg_a, deg_a)

    sim = pl.pallas_call(
        _sim_body, grid=(_G,),
        in_specs=[pl.BlockSpec((_BLK, _F), _row_blk), ne_full],
        out_specs=row_spec,
        out_shape=jax.ShapeDtypeStruct((_N, _N), f32),
    )(ne, ne)

    M1 = pl.pallas_call(
        _topk_body, grid=(_G,),
        in_specs=[row_spec, vec_blk, pl.BlockSpec((_N, _N), _full2)],
        out_specs=row_spec,
        out_shape=jax.ShapeDtypeStruct((_N, _N), f32),
    )(A_ned, deg_n, sim)

    A_fin, deg_f = pl.pallas_call(
        _fin_body, grid=(_G,),
        in_specs=[row_spec, row_spec,
                  pl.BlockSpec((_N, _BLK), lambda i: (0, i))],
        out_specs=(row_spec, pl.BlockSpec((_BLK,), lambda i: (i,))),
        out_shape=(jax.ShapeDtypeStruct((_N, _N), f32),
                   jax.ShapeDtypeStruct((_N,), f32)),
    )(A_raw, M1, M1)

    xs = pl.pallas_call(
        _mid_body,
        out_shape=jax.ShapeDtypeStruct((_N, W1.shape[1]), f32),
    )(deg_f, input, W1)

    h = pl.pallas_call(
        _gcn1_body, grid=(_G,),
        in_specs=[row_spec, vec_blk, pl.BlockSpec((_N, W1.shape[1]), _full2),
                  pl.BlockSpec((W1.shape[1],), _full1)],
        out_specs=pl.BlockSpec((_BLK, W1.shape[1]), _row_blk),
        out_shape=jax.ShapeDtypeStruct((_N, W1.shape[1]), f32),
    )(A_fin, deg_f, xs, b1)

    out = pl.pallas_call(
        _gcn2_body, grid=(_G,),
        in_specs=[row_spec, vec_full, vec_blk,
                  pl.BlockSpec((_N, W1.shape[1]), _full2),
                  pl.BlockSpec((W1.shape[1], b2.shape[0]), _full2),
                  pl.BlockSpec((b2.shape[0],), _full1)],
        out_specs=pl.BlockSpec((_BLK, b2.shape[0]), _row_blk),
        out_shape=jax.ShapeDtypeStruct((_N, b2.shape[0]), f32),
    )(A_fin, deg_f, deg_f, h, W2, b2)
    return out
